# initial kernel scaffold (unmeasured)
import jax
import jax.numpy as jnp
from jax import lax
from jax.experimental import pallas as pl
from jax.experimental.pallas import tpu as pltpu

N_DEV = 4
SQ = 256
SKV = 4096
D_MODEL = 1024
H_PER = 8
DH = 128
SCALE = 0.08838834764831843


def _ag_body(x_ref, out_ref, send_sems, recv_sems):
    my = lax.axis_index("i")
    left = (my - 1) % N_DEV
    right = (my + 1) % N_DEV

    barrier_sem = pltpu.get_barrier_semaphore()
    for nbr in [left, right]:
        pl.semaphore_signal(
            barrier_sem, inc=1,
            device_id=(nbr,), device_id_type=pl.DeviceIdType.MESH,
        )
    pl.semaphore_wait(barrier_sem, 2)

    out_ref[my] = x_ref[0].astype(jnp.bfloat16)

    for h in range(N_DEV - 1):
        o = (my - h) % N_DEV
        rdma = pltpu.make_async_remote_copy(
            src_ref=out_ref.at[o],
            dst_ref=out_ref.at[o],
            send_sem=send_sems.at[h],
            recv_sem=recv_sems.at[h],
            device_id=(right,),
            device_id_type=pl.DeviceIdType.MESH,
        )
        rdma.start()
        rdma.wait()


def _all_gather_x(x):
    return pl.pallas_call(
        _ag_body,
        out_shape=jax.ShapeDtypeStruct((N_DEV, SQ, D_MODEL), jnp.bfloat16),
        in_specs=[pl.BlockSpec(memory_space=pltpu.VMEM)],
        out_specs=pl.BlockSpec(memory_space=pltpu.VMEM),
        scratch_shapes=[
            pltpu.SemaphoreType.DMA((N_DEV - 1,)),
            pltpu.SemaphoreType.DMA((N_DEV - 1,)),
        ],
        compiler_params=pltpu.CompilerParams(collective_id=0),
    )(x)


def _attn_body(x_ref, wq_ref, k_hbm, v_hbm, wo_ref, out_ref,
               kbuf, vbuf, ksems, vsems):
    b = pl.program_id(0)
    my = lax.axis_index("i")
    h0 = my * H_PER

    row = lax.broadcasted_iota(jnp.int32, (SQ, SKV), 0)
    col = lax.broadcasted_iota(jnp.int32, (SQ, SKV), 1)
    qb = row // 64
    kb = col // 64
    mask = (qb == kb) | (kb == 0) | (((qb + kb) % 3) == 0)
    bias = jnp.where(mask, 0.0, -1e9).astype(jnp.float32)

    def start_copies(h, slot):
        hg = h0 + h
        kc = pltpu.make_async_copy(k_hbm.at[b, :, hg, :], kbuf.at[slot],
                                   ksems.at[slot])
        vc = pltpu.make_async_copy(v_hbm.at[b, :, hg, :], vbuf.at[slot],
                                   vsems.at[slot])
        kc.start()
        vc.start()
        return kc, vc

    copies = [None] * H_PER
    copies[0] = start_copies(0, 0)
    acc = None
    for h in range(H_PER):
        slot = h % 2
        if h + 1 < H_PER:
            copies[h + 1] = start_copies(h + 1, (h + 1) % 2)
        kc, vc = copies[h]
        kc.wait()
        vc.wait()

        q = jnp.dot(x_ref[0], wq_ref[:, h * DH:(h + 1) * DH],
                    preferred_element_type=jnp.float32)
        q16 = q.astype(jnp.bfloat16)
        k16 = kbuf[slot].astype(jnp.bfloat16)
        s = lax.dot_general(q16, k16, (((1,), (1,)), ((), ())),
                            preferred_element_type=jnp.float32)
        s = s * SCALE + bias
        e = jnp.exp(s)
        denom = jnp.sum(e, axis=1, keepdims=True)
        e16 = e.astype(jnp.bfloat16)
        v16 = vbuf[slot].astype(jnp.bfloat16)
        ctx = jnp.dot(e16, v16, preferred_element_type=jnp.float32)
        ctx = ctx / denom
        po = jnp.dot(ctx.astype(jnp.bfloat16), wo_ref[h * DH:(h + 1) * DH, :],
                     preferred_element_type=jnp.float32)
        acc = po if acc is None else acc + po
    out_ref[0] = acc


def _attention(x_all16, wq16, k_ext, v_ext, wo16):
    return pl.pallas_call(
        _attn_body,
        grid=(N_DEV,),
        out_shape=jax.ShapeDtypeStruct((N_DEV, SQ, D_MODEL), jnp.float32),
        in_specs=[
            pl.BlockSpec((1, SQ, D_MODEL), lambda b: (b, 0, 0)),
            pl.BlockSpec((D_MODEL, D_MODEL), lambda b: (0, 0)),
            pl.BlockSpec(memory_space=pltpu.ANY),
            pl.BlockSpec(memory_space=pltpu.ANY),
            pl.BlockSpec((D_MODEL, D_MODEL), lambda b: (0, 0)),
        ],
        out_specs=pl.BlockSpec((1, SQ, D_MODEL), lambda b: (b, 0, 0)),
        scratch_shapes=[
            pltpu.VMEM((2, SKV, DH), jnp.float32),
            pltpu.VMEM((2, SKV, DH), jnp.float32),
            pltpu.SemaphoreType.DMA((2,)),
            pltpu.SemaphoreType.DMA((2,)),
        ],
        compiler_params=pltpu.CompilerParams(
            dimension_semantics=("arbitrary",),
        ),
    )(x_all16, wq16, k_ext, v_ext, wo16)


def _rs_body(p_ref, out_ref, comm, sbuf, send_sems, recv_sems):
    my = lax.axis_index("i")
    left = (my - 1) % N_DEV
    right = (my + 1) % N_DEV

    barrier_sem = pltpu.get_barrier_semaphore()
    for nbr in [left, right]:
        pl.semaphore_signal(
            barrier_sem, inc=1,
            device_id=(nbr,), device_id_type=pl.DeviceIdType.MESH,
        )
    pl.semaphore_wait(barrier_sem, 2)

    for s in range(N_DEV - 1):
        cs = (my - 1 - s) % N_DEV
        if s == 0:
            src = p_ref.at[cs]
        else:
            sbuf[s - 1] = comm[s - 1] + p_ref[cs]
            src = sbuf.at[s - 1]
        rdma = pltpu.make_async_remote_copy(
            src_ref=src,
            dst_ref=comm.at[s],
            send_sem=send_sems.at[s],
            recv_sem=recv_sems.at[s],
            device_id=(right,),
            device_id_type=pl.DeviceIdType.MESH,
        )
        rdma.start()
        rdma.wait()

    out_ref[0] = comm[N_DEV - 2] + p_ref[my]


def _reduce_scatter(partial):
    return pl.pallas_call(
        _rs_body,
        out_shape=jax.ShapeDtypeStruct((1, SQ, D_MODEL), jnp.float32),
        in_specs=[pl.BlockSpec(memory_space=pltpu.VMEM)],
        out_specs=pl.BlockSpec(memory_space=pltpu.VMEM),
        scratch_shapes=[
            pltpu.VMEM((N_DEV - 1, SQ, D_MODEL), jnp.float32),
            pltpu.VMEM((N_DEV - 2, SQ, D_MODEL), jnp.float32),
            pltpu.SemaphoreType.DMA((N_DEV - 1,)),
            pltpu.SemaphoreType.DMA((N_DEV - 1,)),
        ],
        compiler_params=pltpu.CompilerParams(collective_id=1),
    )(partial)


def kernel(x, Wq, K_ext, V_ext, Wo):
    x_all16 = _all_gather_x(x)
    partial = _attention(x_all16, Wq.astype(jnp.bfloat16), K_ext, V_ext,
                         Wo.astype(jnp.bfloat16))
    return _reduce_scatter(partial)


# baseline (device time: 160618 ns/iter reference)
import jax
import jax.numpy as jnp
from jax import lax
from jax.experimental import pallas as pl
from jax.experimental.pallas import tpu as pltpu

N_DEV = 4
SQ = 256
SKV = 4096
D_MODEL = 1024
H_PER = 8
DH = 128
SCALE = 0.08838834764831843


def _ag_body(x_ref, out_ref, send_sems, recv_sems):
    my = lax.axis_index("i")
    left = (my - 1) % N_DEV
    right = (my + 1) % N_DEV

    barrier_sem = pltpu.get_barrier_semaphore()
    for nbr in [left, right]:
        pl.semaphore_signal(
            barrier_sem, inc=1,
            device_id=(nbr,), device_id_type=pl.DeviceIdType.MESH,
        )
    pl.semaphore_wait(barrier_sem, 2)

    out_ref[my] = x_ref[0].astype(jnp.bfloat16)

    for h in range(N_DEV - 1):
        o = (my - h) % N_DEV
        rdma = pltpu.make_async_remote_copy(
            src_ref=out_ref.at[o],
            dst_ref=out_ref.at[o],
            send_sem=send_sems.at[h],
            recv_sem=recv_sems.at[h],
            device_id=(right,),
            device_id_type=pl.DeviceIdType.MESH,
        )
        rdma.start()
        rdma.wait()


def _all_gather_x(x):
    return pl.pallas_call(
        _ag_body,
        out_shape=jax.ShapeDtypeStruct((N_DEV, SQ, D_MODEL), jnp.bfloat16),
        in_specs=[pl.BlockSpec(memory_space=pltpu.VMEM)],
        out_specs=pl.BlockSpec(memory_space=pltpu.VMEM),
        scratch_shapes=[
            pltpu.SemaphoreType.DMA((N_DEV - 1,)),
            pltpu.SemaphoreType.DMA((N_DEV - 1,)),
        ],
        compiler_params=pltpu.CompilerParams(collective_id=0),
    )(x)


def _attn_body(x_ref, wq_ref, k_hbm, v_hbm, wo_ref, out_ref,
               kbuf, vbuf, ksems, vsems):
    b = pl.program_id(0)
    my = lax.axis_index("i")
    h0 = my * H_PER

    row = lax.broadcasted_iota(jnp.int32, (SQ, SKV), 0)
    col = lax.broadcasted_iota(jnp.int32, (SQ, SKV), 1)
    qb = row // 64
    kb = col // 64
    mask = (qb == kb) | (kb == 0) | (((qb + kb) % 3) == 0)
    bias = jnp.where(mask, 0.0, -1e9).astype(jnp.float32)

    def start_copies(h, slot):
        hg = h0 + h
        kc = pltpu.make_async_copy(k_hbm.at[b, :, hg, :], kbuf.at[slot],
                                   ksems.at[slot])
        vc = pltpu.make_async_copy(v_hbm.at[b, :, hg, :], vbuf.at[slot],
                                   vsems.at[slot])
        kc.start()
        vc.start()
        return kc, vc

    copies = [None] * H_PER
    copies[0] = start_copies(0, 0)
    acc = None
    for h in range(H_PER):
        slot = h % 2
        if h + 1 < H_PER:
            copies[h + 1] = start_copies(h + 1, (h + 1) % 2)
        kc, vc = copies[h]
        kc.wait()
        vc.wait()

        q = jnp.dot(x_ref[0], wq_ref[:, h * DH:(h + 1) * DH],
                    preferred_element_type=jnp.float32)
        q16 = q.astype(jnp.bfloat16)
        k16 = kbuf[slot].astype(jnp.bfloat16)
        s = lax.dot_general(q16, k16, (((1,), (1,)), ((), ())),
                            preferred_element_type=jnp.float32)
        s = s * SCALE + bias
        e = jnp.exp(s)
        denom = jnp.sum(e, axis=1, keepdims=True)
        e16 = e.astype(jnp.bfloat16)
        v16 = vbuf[slot].astype(jnp.bfloat16)
        ctx = jnp.dot(e16, v16, preferred_element_type=jnp.float32)
        ctx = ctx / denom
        po = jnp.dot(ctx.astype(jnp.bfloat16), wo_ref[h * DH:(h + 1) * DH, :],
                     preferred_element_type=jnp.float32)
        acc = po if acc is None else acc + po
    out_ref[0] = acc


def _attention(x_all16, wq16, k_ext, v_ext, wo16):
    return pl.pallas_call(
        _attn_body,
        grid=(N_DEV,),
        out_shape=jax.ShapeDtypeStruct((N_DEV, SQ, D_MODEL), jnp.float32),
        in_specs=[
            pl.BlockSpec((1, SQ, D_MODEL), lambda b: (b, 0, 0)),
            pl.BlockSpec((D_MODEL, D_MODEL), lambda b: (0, 0)),
            pl.BlockSpec(memory_space=pl.ANY),
            pl.BlockSpec(memory_space=pl.ANY),
            pl.BlockSpec((D_MODEL, D_MODEL), lambda b: (0, 0)),
        ],
        out_specs=pl.BlockSpec((1, SQ, D_MODEL), lambda b: (b, 0, 0)),
        scratch_shapes=[
            pltpu.VMEM((2, SKV, DH), jnp.float32),
            pltpu.VMEM((2, SKV, DH), jnp.float32),
            pltpu.SemaphoreType.DMA((2,)),
            pltpu.SemaphoreType.DMA((2,)),
        ],
        compiler_params=pltpu.CompilerParams(
            dimension_semantics=("arbitrary",),
        ),
    )(x_all16, wq16, k_ext, v_ext, wo16)


def _rs_body(p_ref, out_ref, comm, sbuf, send_sems, recv_sems):
    my = lax.axis_index("i")
    left = (my - 1) % N_DEV
    right = (my + 1) % N_DEV

    barrier_sem = pltpu.get_barrier_semaphore()
    for nbr in [left, right]:
        pl.semaphore_signal(
            barrier_sem, inc=1,
            device_id=(nbr,), device_id_type=pl.DeviceIdType.MESH,
        )
    pl.semaphore_wait(barrier_sem, 2)

    for s in range(N_DEV - 1):
        cs = (my - 1 - s) % N_DEV
        if s == 0:
            src = p_ref.at[cs]
        else:
            sbuf[s - 1] = comm[s - 1] + p_ref[cs]
            src = sbuf.at[s - 1]
        rdma = pltpu.make_async_remote_copy(
            src_ref=src,
            dst_ref=comm.at[s],
            send_sem=send_sems.at[s],
            recv_sem=recv_sems.at[s],
            device_id=(right,),
            device_id_type=pl.DeviceIdType.MESH,
        )
        rdma.start()
        rdma.wait()

    out_ref[0] = comm[N_DEV - 2] + p_ref[my]


def _reduce_scatter(partial):
    return pl.pallas_call(
        _rs_body,
        out_shape=jax.ShapeDtypeStruct((1, SQ, D_MODEL), jnp.float32),
        in_specs=[pl.BlockSpec(memory_space=pltpu.VMEM)],
        out_specs=pl.BlockSpec(memory_space=pltpu.VMEM),
        scratch_shapes=[
            pltpu.VMEM((N_DEV - 1, SQ, D_MODEL), jnp.float32),
            pltpu.VMEM((N_DEV - 2, SQ, D_MODEL), jnp.float32),
            pltpu.SemaphoreType.DMA((N_DEV - 1,)),
            pltpu.SemaphoreType.DMA((N_DEV - 1,)),
        ],
        compiler_params=pltpu.CompilerParams(collective_id=1),
    )(partial)


def kernel(x, Wq, K_ext, V_ext, Wo):
    x_all16 = _all_gather_x(x)
    partial = _attention(x_all16, Wq.astype(jnp.bfloat16), K_ext, V_ext,
                         Wo.astype(jnp.bfloat16))
    return _reduce_scatter(partial)


# device time: 95756 ns/iter; 1.6774x vs baseline; 1.6774x over previous
import jax
import jax.numpy as jnp
from jax import lax
from jax.experimental import pallas as pl
from jax.experimental.pallas import tpu as pltpu

N_DEV = 4
SQ = 256
SKV = 4096
D_MODEL = 1024
H_PER = 8
DH = 128
SCALE = 0.08838834764831843


def _body(x_ref, wq_ref, k_hbm, v_hbm, wo_ref, out_ref,
          xg, kbuf, vbuf, comm, sbuf,
          ag_send, ag_recv, rs_send, rs_recv, ksems, vsems):
    my = lax.axis_index("i")
    left = (my - 1) % N_DEV
    right = (my + 1) % N_DEV

    barrier_sem = pltpu.get_barrier_semaphore()
    for nbr in [left, right]:
        pl.semaphore_signal(
            barrier_sem, inc=1,
            device_id=(nbr,), device_id_type=pl.DeviceIdType.MESH,
        )
    pl.semaphore_wait(barrier_sem, 2)

    row = lax.broadcasted_iota(jnp.int32, (SQ, SKV), 0)
    col = lax.broadcasted_iota(jnp.int32, (SQ, SKV), 1)
    qb = row // 64
    kb = col // 64
    mask = (qb == kb) | (kb == 0) | (((qb + kb) % 3) == 0)
    bias = jnp.where(mask, 0.0, -1e9).astype(jnp.float32)

    xg[my] = x_ref[0].astype(jnp.bfloat16)

    def ag_hop(h):
        o = (my - h) % N_DEV
        rdma = pltpu.make_async_remote_copy(
            src_ref=xg.at[o], dst_ref=xg.at[o],
            send_sem=ag_send.at[h], recv_sem=ag_recv.at[h],
            device_id=(right,), device_id_type=pl.DeviceIdType.MESH,
        )
        rdma.start()
        return rdma

    def rs_hop(s):
        rdma = pltpu.make_async_remote_copy(
            src_ref=sbuf.at[s], dst_ref=comm.at[s],
            send_sem=rs_send.at[s], recv_sem=rs_recv.at[s],
            device_id=(right,), device_id_type=pl.DeviceIdType.MESH,
        )
        rdma.start()
        return rdma

    batches = [my, (my - 1) % N_DEV, (my - 2) % N_DEV, (my + 1) % N_DEV]

    def start_copies(t):
        bi, h = divmod(t, H_PER)
        slot = t % 2
        hg = my * H_PER + h
        b = batches[bi]
        kc = pltpu.make_async_copy(k_hbm.at[b, :, hg, :], kbuf.at[slot],
                                   ksems.at[slot])
        vc = pltpu.make_async_copy(v_hbm.at[b, :, hg, :], vbuf.at[slot],
                                   vsems.at[slot])
        kc.start()
        vc.start()
        return kc, vc

    n_steps = N_DEV * H_PER
    copies = [None] * n_steps
    copies[0] = start_copies(0)

    def head_step(t, x16):
        _, h = divmod(t, H_PER)
        slot = t % 2
        if t + 1 < n_steps:
            copies[t + 1] = start_copies(t + 1)
        kc, vc = copies[t]
        kc.wait()
        vc.wait()
        q = jnp.dot(x16, wq_ref[:, h * DH:(h + 1) * DH],
                    preferred_element_type=jnp.float32)
        q16 = q.astype(jnp.bfloat16)
        k16 = kbuf[slot].astype(jnp.bfloat16)
        s = lax.dot_general(q16, k16, (((1,), (1,)), ((), ())),
                            preferred_element_type=jnp.float32)
        s = s * SCALE + bias
        e = jnp.exp(s)
        denom = jnp.sum(e, axis=1, keepdims=True)
        e16 = e.astype(jnp.bfloat16)
        v16 = vbuf[slot].astype(jnp.bfloat16)
        ctx = jnp.dot(e16, v16, preferred_element_type=jnp.float32)
        ctx = ctx / denom
        return jnp.dot(ctx.astype(jnp.bfloat16),
                       wo_ref[h * DH:(h + 1) * DH, :],
                       preferred_element_type=jnp.float32)

    def batch_partial(bi):
        x16 = xg[batches[bi]]
        acc = None
        for h in range(H_PER):
            po = head_step(bi * H_PER + h, x16)
            acc = po if acc is None else acc + po
        return acc

    ag = [None] * (N_DEV - 1)
    rs = [None] * (N_DEV - 1)

    ag[0] = ag_hop(0)
    p_own = batch_partial(0)

    ag[0].wait_recv()
    ag[1] = ag_hop(1)
    p1 = batch_partial(1)
    sbuf[0] = p1.astype(jnp.bfloat16)
    rs[0] = rs_hop(0)

    ag[1].wait_recv()
    ag[2] = ag_hop(2)
    p2 = batch_partial(2)
    rs[0].wait_recv()
    sbuf[1] = (comm[0].astype(jnp.float32) + p2).astype(jnp.bfloat16)
    rs[1] = rs_hop(1)

    ag[2].wait_recv()
    p3 = batch_partial(3)
    rs[1].wait_recv()
    sbuf[2] = (comm[1].astype(jnp.float32) + p3).astype(jnp.bfloat16)
    rs[2] = rs_hop(2)

    rs[2].wait_recv()
    out_ref[0] = comm[2].astype(jnp.float32) + p_own

    for r in ag + rs:
        r.wait_send()


def kernel(x, Wq, K_ext, V_ext, Wo):
    return pl.pallas_call(
        _body,
        out_shape=jax.ShapeDtypeStruct((1, SQ, D_MODEL), jnp.float32),
        in_specs=[
            pl.BlockSpec(memory_space=pltpu.VMEM),
            pl.BlockSpec(memory_space=pltpu.VMEM),
            pl.BlockSpec(memory_space=pl.ANY),
            pl.BlockSpec(memory_space=pl.ANY),
            pl.BlockSpec(memory_space=pltpu.VMEM),
        ],
        out_specs=pl.BlockSpec(memory_space=pltpu.VMEM),
        scratch_shapes=[
            pltpu.VMEM((N_DEV, SQ, D_MODEL), jnp.bfloat16),
            pltpu.VMEM((2, SKV, DH), jnp.float32),
            pltpu.VMEM((2, SKV, DH), jnp.float32),
            pltpu.VMEM((N_DEV - 1, SQ, D_MODEL), jnp.bfloat16),
            pltpu.VMEM((N_DEV - 1, SQ, D_MODEL), jnp.bfloat16),
            pltpu.SemaphoreType.DMA((N_DEV - 1,)),
            pltpu.SemaphoreType.DMA((N_DEV - 1,)),
            pltpu.SemaphoreType.DMA((N_DEV - 1,)),
            pltpu.SemaphoreType.DMA((N_DEV - 1,)),
            pltpu.SemaphoreType.DMA((2,)),
            pltpu.SemaphoreType.DMA((2,)),
        ],
        compiler_params=pltpu.CompilerParams(collective_id=0),
    )(x, Wq.astype(jnp.bfloat16), K_ext, V_ext, Wo.astype(jnp.bfloat16))
